# baseline (speedup 1.0000x reference)
import functools

import jax
import jax.numpy as jnp
from jax import lax
from jax.experimental import pallas as pl
from jax.experimental.pallas import tpu as pltpu
from jax.experimental.pallas import tpu_sc as plsc

N, E, D = 10000, 320000, 128
R, HR, RD = 32, 64, 32
B, L, T = 64, 3, 2048

NC, NS = 2, 16
NW = NC * NS
EW = E // NW
C = 80
NCH = EW // C
NPAD = 10240
TPT = NPAD // NS
SINK = NPAD - 1
MXCH = 158
MX = MXCH * C

_sc_mesh = plsc.VectorSubcoreMesh(core_axis_name="c", subcore_axis_name="s")
_sc_params = pltpu.CompilerParams(needs_layout_passes=False)

_f32 = jnp.float32


_iota16 = None


@functools.partial(
    pl.kernel,
    out_type=(
        jax.ShapeDtypeStruct((NW * MX,), jnp.int32),
        jax.ShapeDtypeStruct((NW * MX,), jnp.int32),
        jax.ShapeDtypeStruct((NW * MXCH * 16,), jnp.int32),
        jax.ShapeDtypeStruct((NW * 16,), jnp.int32),
        jax.ShapeDtypeStruct((NW, N), _f32),
    ),
    mesh=_sc_mesh,
    compiler_params=_sc_params,
    scratch_types=[
        pltpu.VMEM((EW,), jnp.int32),
        pltpu.VMEM((EW,), jnp.int32),
        pltpu.VMEM((EW,), jnp.int32),
        pltpu.VMEM((N,), _f32),
        pltpu.VMEM((MX,), jnp.int32),
        pltpu.VMEM((MX,), jnp.int32),
        pltpu.VMEM((MXCH * 16,), jnp.int32),
        pltpu.VMEM((R,), jnp.int32),
        pltpu.VMEM((R,), jnp.int32),
        pltpu.VMEM((16,), jnp.int32),
    ],
)
def _pre_sc(src_hbm, dst_hbm, et_hbm, psrc_hbm, pdst_hbm, rl_hbm, nch_hbm,
            deg_hbm, sbuf, dbuf, ebuf, degt, ps, pd, rl, pstart, occ, nchv):
    c = lax.axis_index("c")
    s = lax.axis_index("s")
    wid = c * NS + s
    ebase = wid * EW
    i16 = lax.broadcasted_iota(jnp.int32, (16,), 0)
    zf16 = jnp.zeros((16,), _f32)
    z16 = jnp.zeros((16,), jnp.int32)
    of16 = jnp.ones((16,), _f32)
    o16 = jnp.ones((16,), jnp.int32)

    pltpu.sync_copy(src_hbm.at[pl.ds(ebase, EW)], sbuf)
    pltpu.sync_copy(dst_hbm.at[pl.ds(ebase, EW)], dbuf)
    pltpu.sync_copy(et_hbm.at[pl.ds(ebase, EW)], ebuf)

    def zdeg(i, carry):
        degt[pl.ds(i * 16, 16)] = zf16
        return carry

    lax.fori_loop(0, N // 16, zdeg, 0)
    pstart[pl.ds(0, 16)] = z16
    pstart[pl.ds(16, 16)] = z16
    occ[pl.ds(0, 16)] = z16
    occ[pl.ds(16, 16)] = z16

    def zp(i, carry):
        ps[pl.ds(i * 16, 16)] = (i * 16 + i16) % N
        pd[pl.ds(i * 16, 16)] = N + ((i * 16 + i16) % (NPAD - N))
        return carry

    lax.fori_loop(0, MX // 16, zp, 0)

    def zrl(i, carry):
        rl[pl.ds(i * 16, 16)] = z16
        return carry

    lax.fori_loop(0, MXCH, zrl, 0)

    def hist(i, carry):
        sl = pl.ds(i * 16, 16)
        plsc.addupdate_scatter(degt, [dbuf[sl]], of16)
        plsc.addupdate_scatter(pstart, [ebuf[sl]], o16)
        return carry

    lax.fori_loop(0, EW // 16, hist, 0)

    cnt0 = pstart[pl.ds(0, 16)]
    cnt1 = pstart[pl.ds(16, 16)]
    nchb0 = (cnt0 + (C - 1)) // C
    nchb1 = (cnt1 + (C - 1)) // C
    stch0 = plsc.cumsum(nchb0) - nchb0
    tot0 = jnp.max(plsc.cumsum(nchb0))
    stch1 = plsc.cumsum(nchb1) - nchb1 + tot0
    ncht = jnp.max(plsc.cumsum(nchb1)) + tot0
    ncht = ncht + (ncht % 2)
    pstart[pl.ds(0, 16)] = stch0 * C
    pstart[pl.ds(16, 16)] = stch1 * C
    nchv[...] = jnp.full((16,), 1, jnp.int32) * ncht

    for half, (stch, nchb) in enumerate(((stch0, nchb0), (stch1, nchb1))):
        for rr in range(16):
            r = half * 16 + rr
            m = i16 == rr
            st_r = jnp.max(jnp.where(m, stch, 0))
            nb_r = jnp.max(jnp.where(m, nchb, 0))
            r16 = jnp.full((16,), r, jnp.int32)

            def fill(q, carry):
                rl[pl.ds((st_r + q) * 16, 16)] = r16
                return carry

            lax.fori_loop(0, nb_r, fill, 0)

    def scat(i, carry):
        sl = pl.ds(i * 16, 16)
        e16 = ebuf[sl]
        base = plsc.load_gather(pstart, [e16])
        oc = plsc.load_gather(occ, [e16])
        prior, _ = plsc.scan_count(e16)
        pos = base + oc + prior
        plsc.store_scatter(ps, [pos], sbuf[sl])
        plsc.store_scatter(pd, [pos], dbuf[sl])
        plsc.addupdate_scatter(occ, [e16], o16)
        return carry

    lax.fori_loop(0, EW // 16, scat, 0)

    pltpu.sync_copy(ps, psrc_hbm.at[pl.ds(wid * MX, MX)])
    pltpu.sync_copy(pd, pdst_hbm.at[pl.ds(wid * MX, MX)])
    pltpu.sync_copy(rl, rl_hbm.at[pl.ds(wid * MXCH * 16, MXCH * 16)])
    pltpu.sync_copy(nchv, nch_hbm.at[pl.ds(wid * 16, 16)])
    pltpu.sync_copy(degt, deg_hbm.at[wid])


@functools.partial(
    pl.kernel,
    out_type=jax.ShapeDtypeStruct((NC * NPAD, D), _f32),
    mesh=_sc_mesh,
    compiler_params=_sc_params,
    scratch_types=[
        pltpu.VMEM((C,), jnp.int32),
        pltpu.VMEM((C,), jnp.int32),
        pltpu.VMEM((16,), jnp.int32),
        pltpu.VMEM((C,), jnp.int32),
        pltpu.VMEM((C,), jnp.int32),
        pltpu.VMEM((16,), jnp.int32),
        pltpu.VMEM((C, D), _f32),
        pltpu.VMEM((C, D), _f32),
        pltpu.VMEM((R, D), _f32),
        pltpu.VMEM((16,), jnp.int32),
        pltpu.VMEM_SHARED((NPAD, D), _f32),
        pltpu.SemaphoreType.DMA,
        pltpu.SemaphoreType.DMA,
        pltpu.SemaphoreType.DMA,
        pltpu.SemaphoreType.DMA,
        pltpu.SemaphoreType.DMA,
        pltpu.SemaphoreType.DMA,
        pltpu.SemaphoreType.DMA,
        pltpu.SemaphoreType.DMA,
    ],
)
def _edge_sc(h_hbm, src_hbm, dst_hbm, rl_hbm, nch_hbm, rv_hbm, out_hbm,
             srcA, dstA, relA, srcB, dstB, relB, rowsA, rowsB, rvt, nchv,
             agg_sh, ssemA, dsemA, rsemA, ssemB, dsemB, rsemB, gsemA, gsemB):
    c = lax.axis_index("c")
    s = lax.axis_index("s")
    wid = c * NS + s
    ebase = wid * MX
    cbase = wid * MXCH
    i16 = lax.broadcasted_iota(jnp.int32, (16,), 0)

    def zrow(i, carry):
        rowsA[i // 8, pl.ds((i % 8) * 16, 16)] = jnp.zeros((16,), _f32)
        return carry

    lax.fori_loop(0, C * 8, zrow, 0)

    def zcp(k, carry):
        pltpu.sync_copy(rowsA, agg_sh.at[pl.ds(s * TPT + k * C, C)])
        return carry

    lax.fori_loop(0, TPT // C, zcp, 0)

    pltpu.sync_copy(rv_hbm, rvt)
    pltpu.sync_copy(nch_hbm.at[pl.ds(wid * 16, 16)], nchv)
    nch = MXCH
    plsc.subcore_barrier()

    bufs = (
        (srcA, dstA, relA, rowsA, ssemA, dsemA, rsemA, gsemA),
        (srcB, dstB, relB, rowsB, ssemB, dsemB, rsemB, gsemB),
    )

    def idx_start(i, b):
        src, dst, rel, _, ssem, dsem, rsem, _ = bufs[b]
        off = ebase + i * C
        pltpu.async_copy(src_hbm.at[pl.ds(off, C)], src, ssem)
        pltpu.async_copy(dst_hbm.at[pl.ds(off, C)], dst, dsem)
        pltpu.async_copy(rl_hbm.at[pl.ds((cbase + i) * 16, 16)], rel, rsem)

    def gather_start(b):
        src, _, _, rows, ssem, _, _, gsem = bufs[b]
        pltpu.make_async_copy(src_hbm.at[pl.ds(0, C)], src, ssem).wait()
        pltpu.async_copy(h_hbm.at[src], rows, gsem)

    def mul_scatter(b):
        _, dst, rel, rows, _, dsem, rsem, gsem = bufs[b]
        pltpu.make_async_copy(rl_hbm.at[pl.ds(0, 16)], rel, rsem).wait()
        r16 = rel[...]
        pltpu.make_async_copy(h_hbm.at[pl.ds(0, C)], rows, gsem).wait()

        for k in range(D // 16):
            rvk = plsc.load_gather(rvt, [r16, i16 + k * 16])
            sl = pl.ds(k * 16, 16)

            @plsc.parallel_loop(0, C, 1, unroll=4, carry=rvk)
            def mul(j, rvc):
                rows[j, sl] = rows[j, sl] * rvc
                return rvc

        pltpu.make_async_copy(dst_hbm.at[pl.ds(0, C)], dst, dsem).wait()
        pltpu.sync_copy(rows, agg_sh.at[dst], add=True)

    idx_start(0, 0)
    idx_start(1, 1)
    gather_start(0)

    def pair(t, carry):
        i = t * 2
        gather_start(1)
        mul_scatter(0)

        @pl.when(i + 2 < nch)
        def _():
            idx_start(i + 2, 0)
            gather_start(0)

        mul_scatter(1)

        @pl.when(i + 3 < nch)
        def _():
            idx_start(i + 3, 1)

        return carry

    lax.fori_loop(0, nch // 2, pair, 0)
    plsc.subcore_barrier()
    pltpu.sync_copy(agg_sh.at[pl.ds(s * TPT, TPT)],
                    out_hbm.at[pl.ds(c * NPAD + s * TPT, TPT)])


def _rel_body(srcr_ref, dstr_ref, rel_emb_ref, wrel_ref, wproj_ref, bproj_ref,
              out_ref):
    srcr = srcr_ref[...]
    dstr = dstr_ref[...]
    iota_r = lax.broadcasted_iota(jnp.int32, (R, T), 0)
    ohs = (iota_r == srcr).astype(_f32)
    ohd = (iota_r == dstr).astype(_f32)
    p = lax.dot_general(ohd, ohs, (((1,), (1,)), ((), ())),
                        preferred_element_type=_f32)
    cnt = jnp.sum(p, axis=1, keepdims=True)
    agg = jnp.dot(p, rel_emb_ref[...], preferred_element_type=_f32)
    agg = agg / jnp.maximum(cnt, 1.0)
    emb = jnp.maximum(
        jnp.dot(rel_emb_ref[...] + agg, wrel_ref[...],
                preferred_element_type=_f32), 0.0)
    out_ref[...] = jnp.maximum(
        jnp.dot(emb, wproj_ref[...], preferred_element_type=_f32)
        + bproj_ref[...], 0.0)


_rel_tc = pl.pallas_call(
    _rel_body, out_shape=jax.ShapeDtypeStruct((R, RD), _f32))


def _layer_body(h_ref, a_ref, degp_ref, ws_ref, wn_ref, o_ref):
    deg = jnp.sum(degp_ref[...], axis=1, keepdims=True)
    norm = 1.0 / jnp.maximum(deg, 1.0)
    a = a_ref[...]
    agg = (a[0:N] + a[NPAD:NPAD + N]) * norm
    o_ref[...] = jnp.maximum(
        jnp.dot(h_ref[...], ws_ref[...], preferred_element_type=_f32)
        + jnp.dot(agg, wn_ref[...], preferred_element_type=_f32), 0.0)


_layer_tc = pl.pallas_call(
    _layer_body, out_shape=jax.ShapeDtypeStruct((N, D), _f32))


def _final_body(h1_ref, h2_ref, h3_ref, gid_ref, hid_ref, tid_ref, rl_ref,
                emb_rel_ref, wfc_ref, bfc_ref, out_ref):
    gid = gid_ref[...]
    ohg = (lax.broadcasted_iota(jnp.int32, (B, N), 0) == gid).astype(_f32)
    gcnt = jnp.sum(ohg, axis=1, keepdims=True)
    ginv = 1.0 / jnp.maximum(gcnt, 1.0)
    iota_n = lax.broadcasted_iota(jnp.int32, (B, N), 1)
    ohh = (iota_n == hid_ref[...]).astype(_f32)
    oht = (iota_n == tid_ref[...]).astype(_f32)
    ohr = (lax.broadcasted_iota(jnp.int32, (B, R), 1)
           == rl_ref[...]).astype(_f32)
    wfc = wfc_ref[...]
    bfc = bfc_ref[...]

    hs = (h1_ref[...], h2_ref[...], h3_ref[...])
    acc = jnp.zeros((B, 1), _f32)
    for j in range(L):
        hj = hs[j]
        gj = jnp.dot(ohg, hj, preferred_element_type=_f32) * ginv
        acc = acc + jnp.dot(gj, wfc[j * D:(j + 1) * D],
                            preferred_element_type=_f32)
        hd = jnp.dot(ohh, hj, preferred_element_type=_f32)
        acc = acc + jnp.dot(hd, wfc[L * D + j * D:L * D + (j + 1) * D],
                            preferred_element_type=_f32)
        tl = jnp.dot(oht, hj, preferred_element_type=_f32)
        acc = acc + jnp.dot(tl, wfc[2 * L * D + j * D:2 * L * D + (j + 1) * D],
                            preferred_element_type=_f32)
    emb_sel = jnp.dot(ohr, emb_rel_ref[...], preferred_element_type=_f32)
    acc = acc + jnp.dot(emb_sel, wfc[3 * L * D:3 * L * D + RD],
                        preferred_element_type=_f32)
    out_ref[...] = acc + bfc


_final_tc = pl.pallas_call(
    _final_body, out_shape=jax.ShapeDtypeStruct((B, 1), _f32))


def kernel(x, edge_index, edge_type, node_graph_ids, head_ids, tail_ids,
           rel_labels, relation_triplets, rel_emb, W_rel, W_proj, b_proj,
           rel_vecs, W_self, W_nbr, W_fc, b_fc):
    src = edge_index[0]
    dst = edge_index[1]
    psrc, pdst, rl16, nch16, degp = _pre_sc(src, dst, edge_type)
    degp_t = degp.T

    emb_rel = _rel_tc(relation_triplets[:, 0].reshape(1, T).astype(jnp.int32),
                      relation_triplets[:, 2].reshape(1, T).astype(jnp.int32),
                      rel_emb, W_rel, W_proj, b_proj.reshape(1, RD))

    h = x
    hs = []
    for l in range(L):
        aggp = _edge_sc(h, psrc, pdst, rl16, nch16, rel_vecs[l])
        h = _layer_tc(h, aggp, degp_t, W_self[l], W_nbr[l])
        hs.append(h)

    out = _final_tc(hs[0], hs[1], hs[2],
                    node_graph_ids.reshape(1, N).astype(jnp.int32),
                    head_ids.reshape(B, 1).astype(jnp.int32),
                    tail_ids.reshape(B, 1).astype(jnp.int32),
                    rel_labels.reshape(B, 1).astype(jnp.int32),
                    emb_rel, W_fc, b_fc.reshape(1, 1))
    return out


# speedup vs baseline: 1.1733x; 1.1733x over previous
import functools

import jax
import jax.numpy as jnp
from jax import lax
from jax.experimental import pallas as pl
from jax.experimental.pallas import tpu as pltpu
from jax.experimental.pallas import tpu_sc as plsc

N, E, D = 10000, 320000, 128
R, HR, RD = 32, 64, 32
B, L, T = 64, 3, 2048

NC, NS = 2, 16
NW = NC * NS
EW = E // NW
C = 80
NCH = EW // C
NPAD = 10240
TPT = NPAD // NS
SINK = NPAD - 1
MXCH = 158
MX = MXCH * C

_sc_mesh = plsc.VectorSubcoreMesh(core_axis_name="c", subcore_axis_name="s")
_sc_params = pltpu.CompilerParams(needs_layout_passes=False)

_f32 = jnp.float32


_iota16 = None


@functools.partial(
    pl.kernel,
    out_type=(
        jax.ShapeDtypeStruct((NW * MX,), jnp.int32),
        jax.ShapeDtypeStruct((NW * MX,), jnp.int32),
        jax.ShapeDtypeStruct((NW * MXCH * 16,), jnp.int32),
        jax.ShapeDtypeStruct((NW * 16,), jnp.int32),
        jax.ShapeDtypeStruct((NW, N), _f32),
    ),
    mesh=_sc_mesh,
    compiler_params=_sc_params,
    scratch_types=[
        pltpu.VMEM((EW,), jnp.int32),
        pltpu.VMEM((EW,), jnp.int32),
        pltpu.VMEM((EW,), jnp.int32),
        pltpu.VMEM((N,), _f32),
        pltpu.VMEM((MX,), jnp.int32),
        pltpu.VMEM((MX,), jnp.int32),
        pltpu.VMEM((MXCH * 16,), jnp.int32),
        pltpu.VMEM((R,), jnp.int32),
        pltpu.VMEM((R,), jnp.int32),
        pltpu.VMEM((16,), jnp.int32),
    ],
)
def _pre_sc(src_hbm, dst_hbm, et_hbm, psrc_hbm, pdst_hbm, rl_hbm, nch_hbm,
            deg_hbm, sbuf, dbuf, ebuf, degt, ps, pd, rl, pstart, occ, nchv):
    c = lax.axis_index("c")
    s = lax.axis_index("s")
    wid = c * NS + s
    ebase = wid * EW
    i16 = lax.broadcasted_iota(jnp.int32, (16,), 0)
    zf16 = jnp.zeros((16,), _f32)
    z16 = jnp.zeros((16,), jnp.int32)
    of16 = jnp.ones((16,), _f32)
    o16 = jnp.ones((16,), jnp.int32)

    pltpu.sync_copy(src_hbm.at[pl.ds(ebase, EW)], sbuf)
    pltpu.sync_copy(dst_hbm.at[pl.ds(ebase, EW)], dbuf)
    pltpu.sync_copy(et_hbm.at[pl.ds(ebase, EW)], ebuf)

    def zdeg(i, carry):
        degt[pl.ds(i * 16, 16)] = zf16
        return carry

    lax.fori_loop(0, N // 16, zdeg, 0)
    pstart[pl.ds(0, 16)] = z16
    pstart[pl.ds(16, 16)] = z16
    occ[pl.ds(0, 16)] = z16
    occ[pl.ds(16, 16)] = z16

    def zp(i, carry):
        ps[pl.ds(i * 16, 16)] = (i * 16 + i16) % N
        pd[pl.ds(i * 16, 16)] = N + ((i * 16 + i16) % (NPAD - N))
        return carry

    lax.fori_loop(0, MX // 16, zp, 0)

    def zrl(i, carry):
        rl[pl.ds(i * 16, 16)] = z16
        return carry

    lax.fori_loop(0, MXCH, zrl, 0)

    def hist(i, carry):
        sl = pl.ds(i * 16, 16)
        plsc.addupdate_scatter(degt, [dbuf[sl]], of16)
        plsc.addupdate_scatter(pstart, [ebuf[sl]], o16)
        return carry

    lax.fori_loop(0, EW // 16, hist, 0)

    cnt0 = pstart[pl.ds(0, 16)]
    cnt1 = pstart[pl.ds(16, 16)]
    nchb0 = (cnt0 + (C - 1)) // C
    nchb1 = (cnt1 + (C - 1)) // C
    stch0 = plsc.cumsum(nchb0) - nchb0
    tot0 = jnp.max(plsc.cumsum(nchb0))
    stch1 = plsc.cumsum(nchb1) - nchb1 + tot0
    ncht = jnp.max(plsc.cumsum(nchb1)) + tot0
    ncht = ncht + (ncht % 2)
    pstart[pl.ds(0, 16)] = stch0 * C
    pstart[pl.ds(16, 16)] = stch1 * C
    nchv[...] = jnp.full((16,), 1, jnp.int32) * ncht

    for half, (stch, nchb) in enumerate(((stch0, nchb0), (stch1, nchb1))):
        for rr in range(16):
            r = half * 16 + rr
            m = i16 == rr
            st_r = jnp.max(jnp.where(m, stch, 0))
            nb_r = jnp.max(jnp.where(m, nchb, 0))
            r16 = jnp.full((16,), r, jnp.int32)

            def fill(q, carry):
                rl[pl.ds((st_r + q) * 16, 16)] = r16
                return carry

            lax.fori_loop(0, nb_r, fill, 0)

    def scat(i, carry):
        sl = pl.ds(i * 16, 16)
        e16 = ebuf[sl]
        base = plsc.load_gather(pstart, [e16])
        oc = plsc.load_gather(occ, [e16])
        prior, _ = plsc.scan_count(e16)
        pos = base + oc + prior
        plsc.store_scatter(ps, [pos], sbuf[sl])
        plsc.store_scatter(pd, [pos], dbuf[sl])
        plsc.addupdate_scatter(occ, [e16], o16)
        return carry

    lax.fori_loop(0, EW // 16, scat, 0)

    pltpu.sync_copy(ps, psrc_hbm.at[pl.ds(wid * MX, MX)])
    pltpu.sync_copy(pd, pdst_hbm.at[pl.ds(wid * MX, MX)])
    pltpu.sync_copy(rl, rl_hbm.at[pl.ds(wid * MXCH * 16, MXCH * 16)])
    pltpu.sync_copy(nchv, nch_hbm.at[pl.ds(wid * 16, 16)])
    pltpu.sync_copy(degt, deg_hbm.at[wid])


@functools.partial(
    pl.kernel,
    out_type=jax.ShapeDtypeStruct((NC * NPAD, D), _f32),
    mesh=_sc_mesh,
    compiler_params=_sc_params,
    scratch_types=[
        pltpu.VMEM((C,), jnp.int32),
        pltpu.VMEM((C,), jnp.int32),
        pltpu.VMEM((16,), jnp.int32),
        pltpu.VMEM((C,), jnp.int32),
        pltpu.VMEM((C,), jnp.int32),
        pltpu.VMEM((16,), jnp.int32),
        pltpu.VMEM((C, D), _f32),
        pltpu.VMEM((C, D), _f32),
        pltpu.VMEM((R, D), _f32),
        pltpu.VMEM((16,), jnp.int32),
        pltpu.VMEM_SHARED((NPAD, D), _f32),
        pltpu.SemaphoreType.DMA,
        pltpu.SemaphoreType.DMA,
        pltpu.SemaphoreType.DMA,
        pltpu.SemaphoreType.DMA,
        pltpu.SemaphoreType.DMA,
        pltpu.SemaphoreType.DMA,
        pltpu.SemaphoreType.DMA,
        pltpu.SemaphoreType.DMA,
    ],
)
def _edge_sc(h_hbm, src_hbm, dst_hbm, rl_hbm, nch_hbm, rv_hbm, out_hbm,
             srcA, dstA, relA, srcB, dstB, relB, rowsA, rowsB, rvt, nchv,
             agg_sh, ssemA, dsemA, rsemA, ssemB, dsemB, rsemB, gsemA, gsemB):
    c = lax.axis_index("c")
    s = lax.axis_index("s")
    wid = c * NS + s
    ebase = wid * MX
    cbase = wid * MXCH
    i16 = lax.broadcasted_iota(jnp.int32, (16,), 0)

    def zrow(i, carry):
        rowsA[i // 8, pl.ds((i % 8) * 16, 16)] = jnp.zeros((16,), _f32)
        return carry

    lax.fori_loop(0, C * 8, zrow, 0)

    def zcp(k, carry):
        pltpu.sync_copy(rowsA, agg_sh.at[pl.ds(s * TPT + k * C, C)])
        return carry

    lax.fori_loop(0, TPT // C, zcp, 0)

    pltpu.sync_copy(rv_hbm, rvt)
    pltpu.sync_copy(nch_hbm.at[pl.ds(wid * 16, 16)], nchv)
    nch = MXCH
    plsc.subcore_barrier()

    bufs = (
        (srcA, dstA, relA, rowsA, ssemA, dsemA, rsemA, gsemA),
        (srcB, dstB, relB, rowsB, ssemB, dsemB, rsemB, gsemB),
    )

    def idx_start(i, b):
        src, dst, rel, _, ssem, dsem, rsem, _ = bufs[b]
        off = ebase + i * C
        pltpu.async_copy(src_hbm.at[pl.ds(off, C)], src, ssem)
        pltpu.async_copy(dst_hbm.at[pl.ds(off, C)], dst, dsem)
        pltpu.async_copy(rl_hbm.at[pl.ds((cbase + i) * 16, 16)], rel, rsem)

    def gather_start(b):
        src, _, _, rows, ssem, _, _, gsem = bufs[b]
        pltpu.make_async_copy(src_hbm.at[pl.ds(0, C)], src, ssem).wait()
        pltpu.async_copy(h_hbm.at[src], rows, gsem)

    def mul_scatter(b):
        _, dst, rel, rows, _, dsem, rsem, gsem = bufs[b]
        pltpu.make_async_copy(rl_hbm.at[pl.ds(0, 16)], rel, rsem).wait()
        r16 = rel[...]
        rv = tuple(plsc.load_gather(rvt, [r16, i16 + k * 16])
                   for k in range(D // 16))
        pltpu.make_async_copy(h_hbm.at[pl.ds(0, C)], rows, gsem).wait()

        @plsc.parallel_loop(0, C, 1, unroll=2, carry=rv)
        def mul(j, rvc):
            for k in range(D // 16):
                sl = pl.ds(k * 16, 16)
                rows[j, sl] = rows[j, sl] * rvc[k]
            return rvc

        pltpu.make_async_copy(dst_hbm.at[pl.ds(0, C)], dst, dsem).wait()
        pltpu.sync_copy(rows, agg_sh.at[dst], add=True)

    idx_start(0, 0)
    idx_start(1, 1)
    gather_start(0)

    def pair(t, carry):
        i = t * 2
        gather_start(1)
        mul_scatter(0)

        @pl.when(i + 2 < nch)
        def _():
            idx_start(i + 2, 0)
            gather_start(0)

        mul_scatter(1)

        @pl.when(i + 3 < nch)
        def _():
            idx_start(i + 3, 1)

        return carry

    lax.fori_loop(0, nch // 2, pair, 0)
    plsc.subcore_barrier()
    pltpu.sync_copy(agg_sh.at[pl.ds(s * TPT, TPT)],
                    out_hbm.at[pl.ds(c * NPAD + s * TPT, TPT)])


def _rel_body(srcr_ref, dstr_ref, rel_emb_ref, wrel_ref, wproj_ref, bproj_ref,
              out_ref):
    srcr = srcr_ref[...]
    dstr = dstr_ref[...]
    iota_r = lax.broadcasted_iota(jnp.int32, (R, T), 0)
    ohs = (iota_r == srcr).astype(_f32)
    ohd = (iota_r == dstr).astype(_f32)
    p = lax.dot_general(ohd, ohs, (((1,), (1,)), ((), ())),
                        preferred_element_type=_f32)
    cnt = jnp.sum(p, axis=1, keepdims=True)
    agg = jnp.dot(p, rel_emb_ref[...], preferred_element_type=_f32)
    agg = agg / jnp.maximum(cnt, 1.0)
    emb = jnp.maximum(
        jnp.dot(rel_emb_ref[...] + agg, wrel_ref[...],
                preferred_element_type=_f32), 0.0)
    out_ref[...] = jnp.maximum(
        jnp.dot(emb, wproj_ref[...], preferred_element_type=_f32)
        + bproj_ref[...], 0.0)


_rel_tc = pl.pallas_call(
    _rel_body, out_shape=jax.ShapeDtypeStruct((R, RD), _f32))


def _layer_body(h_ref, a_ref, degp_ref, ws_ref, wn_ref, o_ref):
    deg = jnp.sum(degp_ref[...], axis=1, keepdims=True)
    norm = 1.0 / jnp.maximum(deg, 1.0)
    a = a_ref[...]
    agg = (a[0:N] + a[NPAD:NPAD + N]) * norm
    o_ref[...] = jnp.maximum(
        jnp.dot(h_ref[...], ws_ref[...], preferred_element_type=_f32)
        + jnp.dot(agg, wn_ref[...], preferred_element_type=_f32), 0.0)


_layer_tc = pl.pallas_call(
    _layer_body, out_shape=jax.ShapeDtypeStruct((N, D), _f32))


def _final_body(h1_ref, h2_ref, h3_ref, gid_ref, hid_ref, tid_ref, rl_ref,
                emb_rel_ref, wfc_ref, bfc_ref, out_ref):
    gid = gid_ref[...]
    ohg = (lax.broadcasted_iota(jnp.int32, (B, N), 0) == gid).astype(_f32)
    gcnt = jnp.sum(ohg, axis=1, keepdims=True)
    ginv = 1.0 / jnp.maximum(gcnt, 1.0)
    iota_n = lax.broadcasted_iota(jnp.int32, (B, N), 1)
    ohh = (iota_n == hid_ref[...]).astype(_f32)
    oht = (iota_n == tid_ref[...]).astype(_f32)
    ohr = (lax.broadcasted_iota(jnp.int32, (B, R), 1)
           == rl_ref[...]).astype(_f32)
    wfc = wfc_ref[...]
    bfc = bfc_ref[...]

    hs = (h1_ref[...], h2_ref[...], h3_ref[...])
    acc = jnp.zeros((B, 1), _f32)
    for j in range(L):
        hj = hs[j]
        gj = jnp.dot(ohg, hj, preferred_element_type=_f32) * ginv
        acc = acc + jnp.dot(gj, wfc[j * D:(j + 1) * D],
                            preferred_element_type=_f32)
        hd = jnp.dot(ohh, hj, preferred_element_type=_f32)
        acc = acc + jnp.dot(hd, wfc[L * D + j * D:L * D + (j + 1) * D],
                            preferred_element_type=_f32)
        tl = jnp.dot(oht, hj, preferred_element_type=_f32)
        acc = acc + jnp.dot(tl, wfc[2 * L * D + j * D:2 * L * D + (j + 1) * D],
                            preferred_element_type=_f32)
    emb_sel = jnp.dot(ohr, emb_rel_ref[...], preferred_element_type=_f32)
    acc = acc + jnp.dot(emb_sel, wfc[3 * L * D:3 * L * D + RD],
                        preferred_element_type=_f32)
    out_ref[...] = acc + bfc


_final_tc = pl.pallas_call(
    _final_body, out_shape=jax.ShapeDtypeStruct((B, 1), _f32))


def kernel(x, edge_index, edge_type, node_graph_ids, head_ids, tail_ids,
           rel_labels, relation_triplets, rel_emb, W_rel, W_proj, b_proj,
           rel_vecs, W_self, W_nbr, W_fc, b_fc):
    src = edge_index[0]
    dst = edge_index[1]
    psrc, pdst, rl16, nch16, degp = _pre_sc(src, dst, edge_type)
    degp_t = degp.T

    emb_rel = _rel_tc(relation_triplets[:, 0].reshape(1, T).astype(jnp.int32),
                      relation_triplets[:, 2].reshape(1, T).astype(jnp.int32),
                      rel_emb, W_rel, W_proj, b_proj.reshape(1, RD))

    h = x
    hs = []
    for l in range(L):
        aggp = _edge_sc(h, psrc, pdst, rl16, nch16, rel_vecs[l])
        h = _layer_tc(h, aggp, degp_t, W_self[l], W_nbr[l])
        hs.append(h)

    out = _final_tc(hs[0], hs[1], hs[2],
                    node_graph_ids.reshape(1, N).astype(jnp.int32),
                    head_ids.reshape(B, 1).astype(jnp.int32),
                    tail_ids.reshape(B, 1).astype(jnp.int32),
                    rel_labels.reshape(B, 1).astype(jnp.int32),
                    emb_rel, W_fc, b_fc.reshape(1, 1))
    return out


# speedup vs baseline: 1.6183x; 1.3793x over previous
import functools

import jax
import jax.numpy as jnp
from jax import lax
from jax.experimental import pallas as pl
from jax.experimental.pallas import tpu as pltpu
from jax.experimental.pallas import tpu_sc as plsc

N, E, D = 10000, 320000, 128
R, HR, RD = 32, 64, 32
B, L, T = 64, 3, 2048

NC, NS = 2, 16
NW = NC * NS
EW = E // NW
C = 80
NCH = EW // C
NPAD = 10240
TPT = NPAD // NS
NCHP = 162
MXCH = 168
MX = MXCH * C

_sc_mesh = plsc.VectorSubcoreMesh(core_axis_name="c", subcore_axis_name="s")
_sc_params = pltpu.CompilerParams(needs_layout_passes=False)

_f32 = jnp.float32


_iota16 = None


@functools.partial(
    pl.kernel,
    out_type=(
        jax.ShapeDtypeStruct((NW * MX,), jnp.int32),
        jax.ShapeDtypeStruct((NW * MX,), jnp.int32),
        jax.ShapeDtypeStruct((NW * MXCH * 16,), jnp.int32),
        jax.ShapeDtypeStruct((NW, N), _f32),
    ),
    mesh=_sc_mesh,
    compiler_params=_sc_params,
    scratch_types=[
        pltpu.VMEM((EW,), jnp.int32),
        pltpu.VMEM((EW,), jnp.int32),
        pltpu.VMEM((EW,), jnp.int32),
        pltpu.VMEM((N,), _f32),
        pltpu.VMEM((MX,), jnp.int32),
        pltpu.VMEM((MX,), jnp.int32),
        pltpu.VMEM((MXCH * 16,), jnp.int32),
        pltpu.VMEM((R,), jnp.int32),
        pltpu.VMEM((R,), jnp.int32),
    ],
)
def _pre_sc(src_hbm, dst_hbm, et_hbm, psrc_hbm, pdst_hbm, rl_hbm,
            deg_hbm, sbuf, dbuf, ebuf, degt, ps, pd, rl, pstart, occ):
    c = lax.axis_index("c")
    s = lax.axis_index("s")
    wid = c * NS + s
    ebase = wid * EW
    i16 = lax.broadcasted_iota(jnp.int32, (16,), 0)
    zf16 = jnp.zeros((16,), _f32)
    z16 = jnp.zeros((16,), jnp.int32)
    of16 = jnp.ones((16,), _f32)
    o16 = jnp.ones((16,), jnp.int32)

    pltpu.sync_copy(src_hbm.at[pl.ds(ebase, EW)], sbuf)
    pltpu.sync_copy(dst_hbm.at[pl.ds(ebase, EW)], dbuf)
    pltpu.sync_copy(et_hbm.at[pl.ds(ebase, EW)], ebuf)

    def zdeg(i, carry):
        degt[pl.ds(i * 16, 16)] = zf16
        return carry

    lax.fori_loop(0, N // 16, zdeg, 0)
    pstart[pl.ds(0, 16)] = z16
    pstart[pl.ds(16, 16)] = z16
    occ[pl.ds(0, 16)] = z16
    occ[pl.ds(16, 16)] = z16

    def zp(i, carry):
        ps[pl.ds(i * 16, 16)] = (i * 16 + i16) % N
        pd[pl.ds(i * 16, 16)] = N + ((i * 16 + i16) % (NPAD - N))
        return carry

    lax.fori_loop(0, MX // 16, zp, 0)

    def zrl(i, carry):
        rl[pl.ds(i * 16, 16)] = z16
        return carry

    lax.fori_loop(0, MXCH, zrl, 0)

    def hist(i, carry):
        sl = pl.ds(i * 16, 16)
        plsc.addupdate_scatter(degt, [dbuf[sl]], of16)
        plsc.addupdate_scatter(pstart, [ebuf[sl]], o16)
        return carry

    lax.fori_loop(0, EW // 16, hist, 0)

    cnt0 = pstart[pl.ds(0, 16)]
    cnt1 = pstart[pl.ds(16, 16)]
    nchb0 = (cnt0 + (C - 1)) // C
    nchb1 = (cnt1 + (C - 1)) // C
    stch0 = plsc.cumsum(nchb0) - nchb0
    tot0 = jnp.max(plsc.cumsum(nchb0))
    stch1 = plsc.cumsum(nchb1) - nchb1 + tot0
    pstart[pl.ds(0, 16)] = stch0 * C
    pstart[pl.ds(16, 16)] = stch1 * C

    for half, (stch, nchb) in enumerate(((stch0, nchb0), (stch1, nchb1))):
        for rr in range(16):
            r = half * 16 + rr
            m = i16 == rr
            st_r = jnp.max(jnp.where(m, stch, 0))
            nb_r = jnp.max(jnp.where(m, nchb, 0))
            r16 = jnp.full((16,), r, jnp.int32)

            def fill(q, carry):
                rl[pl.ds((st_r + q) * 16, 16)] = r16
                return carry

            lax.fori_loop(0, nb_r, fill, 0)

    def scat(i, carry):
        sl = pl.ds(i * 16, 16)
        e16 = ebuf[sl]
        base = plsc.load_gather(pstart, [e16])
        oc = plsc.load_gather(occ, [e16])
        prior, _ = plsc.scan_count(e16)
        pos = base + oc + prior
        plsc.store_scatter(ps, [pos], sbuf[sl])
        plsc.store_scatter(pd, [pos], dbuf[sl])
        plsc.addupdate_scatter(occ, [e16], o16)
        return carry

    lax.fori_loop(0, EW // 16, scat, 0)

    pltpu.sync_copy(ps, psrc_hbm.at[pl.ds(wid * MX, MX)])
    pltpu.sync_copy(pd, pdst_hbm.at[pl.ds(wid * MX, MX)])
    pltpu.sync_copy(rl, rl_hbm.at[pl.ds(wid * MXCH * 16, MXCH * 16)])
    pltpu.sync_copy(degt, deg_hbm.at[wid])


@functools.partial(
    pl.kernel,
    out_type=jax.ShapeDtypeStruct((NC * NPAD, D), _f32),
    mesh=_sc_mesh,
    compiler_params=_sc_params,
    scratch_types=[
        pltpu.VMEM((3, C), jnp.int32),
        pltpu.VMEM((3, C), jnp.int32),
        pltpu.VMEM((3, 16), jnp.int32),
        pltpu.VMEM((C, D), _f32),
        pltpu.VMEM((C, D), _f32),
        pltpu.VMEM((C, D), _f32),
        pltpu.VMEM((R, D), _f32),
        pltpu.VMEM_SHARED((NPAD, D), _f32),
        pltpu.SemaphoreType.DMA,
        pltpu.SemaphoreType.DMA,
        pltpu.SemaphoreType.DMA,
        pltpu.SemaphoreType.DMA,
        pltpu.SemaphoreType.DMA,
        pltpu.SemaphoreType.DMA,
        pltpu.SemaphoreType.DMA,
        pltpu.SemaphoreType.DMA,
        pltpu.SemaphoreType.DMA,
        pltpu.SemaphoreType.DMA,
        pltpu.SemaphoreType.DMA,
        pltpu.SemaphoreType.DMA,
        pltpu.SemaphoreType.DMA,
        pltpu.SemaphoreType.DMA,
        pltpu.SemaphoreType.DMA,
    ],
)
def _edge_sc(h_hbm, src_hbm, dst_hbm, rl_hbm, rv_hbm, out_hbm,
             srci, dsti, reli, rows0, rows1, rows2, rvt, agg_sh,
             ssem0, ssem1, ssem2, dsem0, dsem1, dsem2,
             rsem0, rsem1, rsem2, gsem0, gsem1, gsem2,
             csem0, csem1, csem2):
    c = lax.axis_index("c")
    s = lax.axis_index("s")
    wid = c * NS + s
    ebase = wid * MX
    cbase = wid * MXCH
    i16 = lax.broadcasted_iota(jnp.int32, (16,), 0)

    def zrow(i, carry):
        rows0[i // 8, pl.ds((i % 8) * 16, 16)] = jnp.zeros((16,), _f32)
        return carry

    lax.fori_loop(0, C * 8, zrow, 0)

    def zcp(k, carry):
        pltpu.sync_copy(rows0, agg_sh.at[pl.ds(s * TPT + k * C, C)])
        return carry

    lax.fori_loop(0, TPT // C, zcp, 0)

    pltpu.sync_copy(rv_hbm, rvt)
    plsc.subcore_barrier()

    rowsb = (rows0, rows1, rows2)
    ssems = (ssem0, ssem1, ssem2)
    dsems = (dsem0, dsem1, dsem2)
    rsems = (rsem0, rsem1, rsem2)
    gsems = (gsem0, gsem1, gsem2)
    csems = (csem0, csem1, csem2)

    def isrc_start(i, p):
        pltpu.async_copy(src_hbm.at[pl.ds(ebase + i * C, C)],
                         srci.at[p], ssems[p])
        pltpu.async_copy(rl_hbm.at[pl.ds((cbase + i) * 16, 16)],
                         reli.at[p], rsems[p])

    def idst_start(i, p):
        pltpu.async_copy(dst_hbm.at[pl.ds(ebase + i * C, C)],
                         dsti.at[p], dsems[p])

    def gather_start(p):
        pltpu.make_async_copy(src_hbm.at[pl.ds(0, C)],
                              srci.at[p], ssems[p]).wait()
        pltpu.make_async_copy(rl_hbm.at[pl.ds(0, 16)],
                              reli.at[p], rsems[p]).wait()
        pltpu.async_copy(h_hbm.at[srci.at[p]], rowsb[p], gsems[p])

    def scat_wait(p):
        pltpu.make_async_copy(h_hbm.at[pl.ds(0, C)],
                              rowsb[p], csems[p]).wait()

    def mul(p):
        r16 = reli[p, :]
        rv = tuple(plsc.load_gather(rvt, [r16, i16 + k * 16])
                   for k in range(D // 16))
        rows = rowsb[p]
        pltpu.make_async_copy(h_hbm.at[pl.ds(0, C)], rows, gsems[p]).wait()

        @plsc.parallel_loop(0, C, 1, unroll=2, carry=rv)
        def body(j, rvc):
            for k in range(D // 16):
                sl = pl.ds(k * 16, 16)
                rows[j, sl] = rows[j, sl] * rvc[k]
            return rvc

    def scat_start(p):
        pltpu.make_async_copy(dst_hbm.at[pl.ds(0, C)],
                              dsti.at[p], dsems[p]).wait()
        pltpu.async_copy(rowsb[p], agg_sh.at[dsti.at[p]], csems[p])

    for p in range(3):
        isrc_start(p, p)
    idst_start(0, 0)
    gather_start(0)

    def slot(t, p, first=False, last=False):
        cidx = t * 3 + p
        q = (p + 1) % 3
        if not first:
            scat_wait(q)
        if not (last and p == 2):
            idst_start(cidx + 1, q)
            gather_start(q)
        mul(p)
        scat_start(p)
        if not last:
            isrc_start(cidx + 3, p)

    def rot(t, carry):
        for p in range(3):
            slot(t, p)
        return carry

    for p in range(3):
        slot(0, p, first=p < 2)
    lax.fori_loop(1, NCHP // 3 - 1, rot, 0)
    for p in range(3):
        slot(NCHP // 3 - 1, p, last=True)
    scat_wait(1)
    scat_wait(2)
    plsc.subcore_barrier()
    pltpu.sync_copy(agg_sh.at[pl.ds(s * TPT, TPT)],
                    out_hbm.at[pl.ds(c * NPAD + s * TPT, TPT)])


def _rel_body(srcr_ref, dstr_ref, rel_emb_ref, wrel_ref, wproj_ref, bproj_ref,
              out_ref):
    srcr = srcr_ref[...]
    dstr = dstr_ref[...]
    iota_r = lax.broadcasted_iota(jnp.int32, (R, T), 0)
    ohs = (iota_r == srcr).astype(_f32)
    ohd = (iota_r == dstr).astype(_f32)
    p = lax.dot_general(ohd, ohs, (((1,), (1,)), ((), ())),
                        preferred_element_type=_f32)
    cnt = jnp.sum(p, axis=1, keepdims=True)
    agg = jnp.dot(p, rel_emb_ref[...], preferred_element_type=_f32)
    agg = agg / jnp.maximum(cnt, 1.0)
    emb = jnp.maximum(
        jnp.dot(rel_emb_ref[...] + agg, wrel_ref[...],
                preferred_element_type=_f32), 0.0)
    out_ref[...] = jnp.maximum(
        jnp.dot(emb, wproj_ref[...], preferred_element_type=_f32)
        + bproj_ref[...], 0.0)


_rel_tc = pl.pallas_call(
    _rel_body, out_shape=jax.ShapeDtypeStruct((R, RD), _f32))


def _layer_body(h_ref, a_ref, degp_ref, ws_ref, wn_ref, o_ref):
    deg = jnp.sum(degp_ref[...], axis=1, keepdims=True)
    norm = 1.0 / jnp.maximum(deg, 1.0)
    a = a_ref[...]
    agg = (a[0:N] + a[NPAD:NPAD + N]) * norm
    o_ref[...] = jnp.maximum(
        jnp.dot(h_ref[...], ws_ref[...], preferred_element_type=_f32)
        + jnp.dot(agg, wn_ref[...], preferred_element_type=_f32), 0.0)


_layer_tc = pl.pallas_call(
    _layer_body, out_shape=jax.ShapeDtypeStruct((N, D), _f32))


def _final_body(h1_ref, h2_ref, h3_ref, gid_ref, hid_ref, tid_ref, rl_ref,
                emb_rel_ref, wfc_ref, bfc_ref, out_ref):
    gid = gid_ref[...]
    ohg = (lax.broadcasted_iota(jnp.int32, (B, N), 0) == gid).astype(_f32)
    gcnt = jnp.sum(ohg, axis=1, keepdims=True)
    ginv = 1.0 / jnp.maximum(gcnt, 1.0)
    iota_n = lax.broadcasted_iota(jnp.int32, (B, N), 1)
    ohh = (iota_n == hid_ref[...]).astype(_f32)
    oht = (iota_n == tid_ref[...]).astype(_f32)
    ohr = (lax.broadcasted_iota(jnp.int32, (B, R), 1)
           == rl_ref[...]).astype(_f32)
    wfc = wfc_ref[...]
    bfc = bfc_ref[...]

    hs = (h1_ref[...], h2_ref[...], h3_ref[...])
    acc = jnp.zeros((B, 1), _f32)
    for j in range(L):
        hj = hs[j]
        gj = jnp.dot(ohg, hj, preferred_element_type=_f32) * ginv
        acc = acc + jnp.dot(gj, wfc[j * D:(j + 1) * D],
                            preferred_element_type=_f32)
        hd = jnp.dot(ohh, hj, preferred_element_type=_f32)
        acc = acc + jnp.dot(hd, wfc[L * D + j * D:L * D + (j + 1) * D],
                            preferred_element_type=_f32)
        tl = jnp.dot(oht, hj, preferred_element_type=_f32)
        acc = acc + jnp.dot(tl, wfc[2 * L * D + j * D:2 * L * D + (j + 1) * D],
                            preferred_element_type=_f32)
    emb_sel = jnp.dot(ohr, emb_rel_ref[...], preferred_element_type=_f32)
    acc = acc + jnp.dot(emb_sel, wfc[3 * L * D:3 * L * D + RD],
                        preferred_element_type=_f32)
    out_ref[...] = acc + bfc


_final_tc = pl.pallas_call(
    _final_body, out_shape=jax.ShapeDtypeStruct((B, 1), _f32))


def kernel(x, edge_index, edge_type, node_graph_ids, head_ids, tail_ids,
           rel_labels, relation_triplets, rel_emb, W_rel, W_proj, b_proj,
           rel_vecs, W_self, W_nbr, W_fc, b_fc):
    src = edge_index[0]
    dst = edge_index[1]
    psrc, pdst, rl16, degp = _pre_sc(src, dst, edge_type)
    degp_t = degp.T

    emb_rel = _rel_tc(relation_triplets[:, 0].reshape(1, T).astype(jnp.int32),
                      relation_triplets[:, 2].reshape(1, T).astype(jnp.int32),
                      rel_emb, W_rel, W_proj, b_proj.reshape(1, RD))

    h = x
    hs = []
    for l in range(L):
        aggp = _edge_sc(h, psrc, pdst, rl16, rel_vecs[l])
        h = _layer_tc(h, aggp, degp_t, W_self[l], W_nbr[l])
        hs.append(h)

    out = _final_tc(hs[0], hs[1], hs[2],
                    node_graph_ids.reshape(1, N).astype(jnp.int32),
                    head_ids.reshape(B, 1).astype(jnp.int32),
                    tail_ids.reshape(B, 1).astype(jnp.int32),
                    rel_labels.reshape(B, 1).astype(jnp.int32),
                    emb_rel, W_fc, b_fc.reshape(1, 1))
    return out


# speedup vs baseline: 1.6187x; 1.0003x over previous
import functools

import jax
import jax.numpy as jnp
from jax import lax
from jax.experimental import pallas as pl
from jax.experimental.pallas import tpu as pltpu
from jax.experimental.pallas import tpu_sc as plsc

N, E, D = 10000, 320000, 128
R, HR, RD = 32, 64, 32
B, L, T = 64, 3, 2048

NC, NS = 2, 16
NW = NC * NS
EW = E // NW
C = 80
NCH = EW // C
NPAD = 10240
TPT = NPAD // NS
NCHP = 162
MXCH = 168
MX = MXCH * C

_sc_mesh = plsc.VectorSubcoreMesh(core_axis_name="c", subcore_axis_name="s")
_sc_params = pltpu.CompilerParams(needs_layout_passes=False)

_f32 = jnp.float32


_iota16 = None


@functools.partial(
    pl.kernel,
    out_type=(
        jax.ShapeDtypeStruct((NW * MX,), jnp.int32),
        jax.ShapeDtypeStruct((NW * MX,), jnp.int32),
        jax.ShapeDtypeStruct((NW * MXCH * 16,), jnp.int32),
        jax.ShapeDtypeStruct((NW, N), _f32),
    ),
    mesh=_sc_mesh,
    compiler_params=_sc_params,
    scratch_types=[
        pltpu.VMEM((EW,), jnp.int32),
        pltpu.VMEM((EW,), jnp.int32),
        pltpu.VMEM((EW,), jnp.int32),
        pltpu.VMEM((N,), _f32),
        pltpu.VMEM((MX,), jnp.int32),
        pltpu.VMEM((MX,), jnp.int32),
        pltpu.VMEM((MXCH * 16,), jnp.int32),
        pltpu.VMEM((R,), jnp.int32),
        pltpu.VMEM((R,), jnp.int32),
    ],
)
def _pre_sc(src_hbm, dst_hbm, et_hbm, psrc_hbm, pdst_hbm, rl_hbm,
            deg_hbm, sbuf, dbuf, ebuf, degt, ps, pd, rl, pstart, occ):
    c = lax.axis_index("c")
    s = lax.axis_index("s")
    wid = c * NS + s
    ebase = wid * EW
    i16 = lax.broadcasted_iota(jnp.int32, (16,), 0)
    zf16 = jnp.zeros((16,), _f32)
    z16 = jnp.zeros((16,), jnp.int32)
    of16 = jnp.ones((16,), _f32)
    o16 = jnp.ones((16,), jnp.int32)

    pltpu.sync_copy(src_hbm.at[pl.ds(ebase, EW)], sbuf)
    pltpu.sync_copy(dst_hbm.at[pl.ds(ebase, EW)], dbuf)
    pltpu.sync_copy(et_hbm.at[pl.ds(ebase, EW)], ebuf)

    def zdeg(i, carry):
        degt[pl.ds(i * 16, 16)] = zf16
        return carry

    lax.fori_loop(0, N // 16, zdeg, 0)
    pstart[pl.ds(0, 16)] = z16
    pstart[pl.ds(16, 16)] = z16
    occ[pl.ds(0, 16)] = z16
    occ[pl.ds(16, 16)] = z16

    def zp(i, carry):
        ps[pl.ds(i * 16, 16)] = (i * 16 + i16) % N
        pd[pl.ds(i * 16, 16)] = N + ((i * 16 + i16) % (NPAD - N))
        return carry

    lax.fori_loop(0, MX // 16, zp, 0)

    def zrl(i, carry):
        rl[pl.ds(i * 16, 16)] = z16
        return carry

    lax.fori_loop(0, MXCH, zrl, 0)

    def hist(i, carry):
        sl = pl.ds(i * 16, 16)
        plsc.addupdate_scatter(degt, [dbuf[sl]], of16)
        plsc.addupdate_scatter(pstart, [ebuf[sl]], o16)
        return carry

    lax.fori_loop(0, EW // 16, hist, 0)

    cnt0 = pstart[pl.ds(0, 16)]
    cnt1 = pstart[pl.ds(16, 16)]
    nchb0 = (cnt0 + (C - 1)) // C
    nchb1 = (cnt1 + (C - 1)) // C
    stch0 = plsc.cumsum(nchb0) - nchb0
    tot0 = jnp.max(plsc.cumsum(nchb0))
    stch1 = plsc.cumsum(nchb1) - nchb1 + tot0
    pstart[pl.ds(0, 16)] = stch0 * C
    pstart[pl.ds(16, 16)] = stch1 * C

    for half, (stch, nchb) in enumerate(((stch0, nchb0), (stch1, nchb1))):
        for rr in range(16):
            r = half * 16 + rr
            m = i16 == rr
            st_r = jnp.max(jnp.where(m, stch, 0))
            nb_r = jnp.max(jnp.where(m, nchb, 0))
            r16 = jnp.full((16,), r, jnp.int32)

            def fill(q, carry):
                rl[pl.ds((st_r + q) * 16, 16)] = r16
                return carry

            lax.fori_loop(0, nb_r, fill, 0)

    def scat(i, carry):
        sl = pl.ds(i * 16, 16)
        e16 = ebuf[sl]
        base = plsc.load_gather(pstart, [e16])
        oc = plsc.load_gather(occ, [e16])
        prior, _ = plsc.scan_count(e16)
        pos = base + oc + prior
        plsc.store_scatter(ps, [pos], sbuf[sl])
        plsc.store_scatter(pd, [pos], dbuf[sl])
        plsc.addupdate_scatter(occ, [e16], o16)
        return carry

    lax.fori_loop(0, EW // 16, scat, 0)

    pltpu.sync_copy(ps, psrc_hbm.at[pl.ds(wid * MX, MX)])
    pltpu.sync_copy(pd, pdst_hbm.at[pl.ds(wid * MX, MX)])
    pltpu.sync_copy(rl, rl_hbm.at[pl.ds(wid * MXCH * 16, MXCH * 16)])
    pltpu.sync_copy(degt, deg_hbm.at[wid])


@functools.partial(
    pl.kernel,
    out_type=jax.ShapeDtypeStruct((NC * NPAD, D), _f32),
    mesh=_sc_mesh,
    compiler_params=_sc_params,
    scratch_types=[
        pltpu.VMEM((3, C), jnp.int32),
        pltpu.VMEM((3, C), jnp.int32),
        pltpu.VMEM((3, 16), jnp.int32),
        pltpu.VMEM((C, D), _f32),
        pltpu.VMEM((C, D), _f32),
        pltpu.VMEM((C, D), _f32),
        pltpu.VMEM((R, D), _f32),
        pltpu.VMEM_SHARED((NPAD, D), _f32),
        pltpu.SemaphoreType.DMA,
        pltpu.SemaphoreType.DMA,
        pltpu.SemaphoreType.DMA,
        pltpu.SemaphoreType.DMA,
        pltpu.SemaphoreType.DMA,
        pltpu.SemaphoreType.DMA,
        pltpu.SemaphoreType.DMA,
        pltpu.SemaphoreType.DMA,
        pltpu.SemaphoreType.DMA,
        pltpu.SemaphoreType.DMA,
        pltpu.SemaphoreType.DMA,
        pltpu.SemaphoreType.DMA,
        pltpu.SemaphoreType.DMA,
        pltpu.SemaphoreType.DMA,
        pltpu.SemaphoreType.DMA,
    ],
)
def _edge_sc(h_hbm, src_hbm, dst_hbm, rl_hbm, rv_hbm, out_hbm,
             srci, dsti, reli, rows0, rows1, rows2, rvt, agg_sh,
             ssem0, ssem1, ssem2, dsem0, dsem1, dsem2,
             rsem0, rsem1, rsem2, gsem0, gsem1, gsem2,
             csem0, csem1, csem2):
    c = lax.axis_index("c")
    s = lax.axis_index("s")
    wid = c * NS + s
    ebase = wid * MX
    cbase = wid * MXCH
    i16 = lax.broadcasted_iota(jnp.int32, (16,), 0)

    def zrow(i, carry):
        rows0[i // 8, pl.ds((i % 8) * 16, 16)] = jnp.zeros((16,), _f32)
        return carry

    lax.fori_loop(0, C * 8, zrow, 0)

    def zcp(k, carry):
        pltpu.sync_copy(rows0, agg_sh.at[pl.ds(s * TPT + k * C, C)])
        return carry

    lax.fori_loop(0, TPT // C, zcp, 0)

    pltpu.sync_copy(rv_hbm, rvt)
    plsc.subcore_barrier()

    rowsb = (rows0, rows1, rows2)
    ssems = (ssem0, ssem1, ssem2)
    dsems = (dsem0, dsem1, dsem2)
    rsems = (rsem0, rsem1, rsem2)
    gsems = (gsem0, gsem1, gsem2)
    csems = (csem0, csem1, csem2)

    def isrc_start(i, p):
        pltpu.async_copy(src_hbm.at[pl.ds(ebase + i * C, C)],
                         srci.at[p], ssems[p])
        pltpu.async_copy(rl_hbm.at[pl.ds((cbase + i) * 16, 16)],
                         reli.at[p], rsems[p])

    def idst_start(i, p):
        pltpu.async_copy(dst_hbm.at[pl.ds(ebase + i * C, C)],
                         dsti.at[p], dsems[p])

    def gather_start(p):
        pltpu.make_async_copy(src_hbm.at[pl.ds(0, C)],
                              srci.at[p], ssems[p]).wait()
        pltpu.make_async_copy(rl_hbm.at[pl.ds(0, 16)],
                              reli.at[p], rsems[p]).wait()
        pltpu.async_copy(h_hbm.at[srci.at[p]], rowsb[p], gsems[p])

    def scat_wait(p):
        pltpu.make_async_copy(h_hbm.at[pl.ds(0, C)],
                              rowsb[p], csems[p]).wait()

    def mul(p):
        r16 = reli[p, :]
        rv = tuple(plsc.load_gather(rvt, [r16, i16 + k * 16])
                   for k in range(D // 16))
        rows = rowsb[p]
        pltpu.make_async_copy(h_hbm.at[pl.ds(0, C)], rows, gsems[p]).wait()

        @plsc.parallel_loop(0, C, 1, unroll=2, carry=rv)
        def body(j, rvc):
            for k in range(D // 16):
                sl = pl.ds(k * 16, 16)
                rows[j, sl] = rows[j, sl] * rvc[k]
            return rvc

    def scat_start(p):
        pltpu.make_async_copy(dst_hbm.at[pl.ds(0, C)],
                              dsti.at[p], dsems[p]).wait()
        pltpu.async_copy(rowsb[p], agg_sh.at[dsti.at[p]], csems[p])

    for p in range(3):
        isrc_start(p, p)
    idst_start(0, 0)
    gather_start(0)

    def slot(t, p, first=False, last=False):
        cidx = t * 3 + p
        q = (p + 1) % 3
        if not (last and p == 2):
            idst_start(cidx + 1, q)
            gather_start(q)
        mul(p)
        if not first:
            scat_wait((p + 2) % 3)
        scat_start(p)
        if not last:
            isrc_start(cidx + 3, p)

    def rot(t, carry):
        for p in range(3):
            slot(t, p)
        return carry

    for p in range(3):
        slot(0, p, first=p == 0)
    lax.fori_loop(1, NCHP // 3 - 1, rot, 0)
    for p in range(3):
        slot(NCHP // 3 - 1, p, last=True)
    scat_wait(2)
    plsc.subcore_barrier()
    pltpu.sync_copy(agg_sh.at[pl.ds(s * TPT, TPT)],
                    out_hbm.at[pl.ds(c * NPAD + s * TPT, TPT)])


def _rel_body(srcr_ref, dstr_ref, rel_emb_ref, wrel_ref, wproj_ref, bproj_ref,
              out_ref):
    srcr = srcr_ref[...]
    dstr = dstr_ref[...]
    iota_r = lax.broadcasted_iota(jnp.int32, (R, T), 0)
    ohs = (iota_r == srcr).astype(_f32)
    ohd = (iota_r == dstr).astype(_f32)
    p = lax.dot_general(ohd, ohs, (((1,), (1,)), ((), ())),
                        preferred_element_type=_f32)
    cnt = jnp.sum(p, axis=1, keepdims=True)
    agg = jnp.dot(p, rel_emb_ref[...], preferred_element_type=_f32)
    agg = agg / jnp.maximum(cnt, 1.0)
    emb = jnp.maximum(
        jnp.dot(rel_emb_ref[...] + agg, wrel_ref[...],
                preferred_element_type=_f32), 0.0)
    out_ref[...] = jnp.maximum(
        jnp.dot(emb, wproj_ref[...], preferred_element_type=_f32)
        + bproj_ref[...], 0.0)


_rel_tc = pl.pallas_call(
    _rel_body, out_shape=jax.ShapeDtypeStruct((R, RD), _f32))


def _layer_body(h_ref, a_ref, degp_ref, ws_ref, wn_ref, o_ref):
    deg = jnp.sum(degp_ref[...], axis=1, keepdims=True)
    norm = 1.0 / jnp.maximum(deg, 1.0)
    a = a_ref[...]
    agg = (a[0:N] + a[NPAD:NPAD + N]) * norm
    o_ref[...] = jnp.maximum(
        jnp.dot(h_ref[...], ws_ref[...], preferred_element_type=_f32)
        + jnp.dot(agg, wn_ref[...], preferred_element_type=_f32), 0.0)


_layer_tc = pl.pallas_call(
    _layer_body, out_shape=jax.ShapeDtypeStruct((N, D), _f32))


def _final_body(h1_ref, h2_ref, h3_ref, gid_ref, hid_ref, tid_ref, rl_ref,
                emb_rel_ref, wfc_ref, bfc_ref, out_ref):
    gid = gid_ref[...]
    ohg = (lax.broadcasted_iota(jnp.int32, (B, N), 0) == gid).astype(_f32)
    gcnt = jnp.sum(ohg, axis=1, keepdims=True)
    ginv = 1.0 / jnp.maximum(gcnt, 1.0)
    iota_n = lax.broadcasted_iota(jnp.int32, (B, N), 1)
    ohh = (iota_n == hid_ref[...]).astype(_f32)
    oht = (iota_n == tid_ref[...]).astype(_f32)
    ohr = (lax.broadcasted_iota(jnp.int32, (B, R), 1)
           == rl_ref[...]).astype(_f32)
    wfc = wfc_ref[...]
    bfc = bfc_ref[...]

    hs = (h1_ref[...], h2_ref[...], h3_ref[...])
    acc = jnp.zeros((B, 1), _f32)
    for j in range(L):
        hj = hs[j]
        gj = jnp.dot(ohg, hj, preferred_element_type=_f32) * ginv
        acc = acc + jnp.dot(gj, wfc[j * D:(j + 1) * D],
                            preferred_element_type=_f32)
        hd = jnp.dot(ohh, hj, preferred_element_type=_f32)
        acc = acc + jnp.dot(hd, wfc[L * D + j * D:L * D + (j + 1) * D],
                            preferred_element_type=_f32)
        tl = jnp.dot(oht, hj, preferred_element_type=_f32)
        acc = acc + jnp.dot(tl, wfc[2 * L * D + j * D:2 * L * D + (j + 1) * D],
                            preferred_element_type=_f32)
    emb_sel = jnp.dot(ohr, emb_rel_ref[...], preferred_element_type=_f32)
    acc = acc + jnp.dot(emb_sel, wfc[3 * L * D:3 * L * D + RD],
                        preferred_element_type=_f32)
    out_ref[...] = acc + bfc


_final_tc = pl.pallas_call(
    _final_body, out_shape=jax.ShapeDtypeStruct((B, 1), _f32))


def kernel(x, edge_index, edge_type, node_graph_ids, head_ids, tail_ids,
           rel_labels, relation_triplets, rel_emb, W_rel, W_proj, b_proj,
           rel_vecs, W_self, W_nbr, W_fc, b_fc):
    src = edge_index[0]
    dst = edge_index[1]
    psrc, pdst, rl16, degp = _pre_sc(src, dst, edge_type)
    degp_t = degp.T

    emb_rel = _rel_tc(relation_triplets[:, 0].reshape(1, T).astype(jnp.int32),
                      relation_triplets[:, 2].reshape(1, T).astype(jnp.int32),
                      rel_emb, W_rel, W_proj, b_proj.reshape(1, RD))

    h = x
    hs = []
    for l in range(L):
        aggp = _edge_sc(h, psrc, pdst, rl16, rel_vecs[l])
        h = _layer_tc(h, aggp, degp_t, W_self[l], W_nbr[l])
        hs.append(h)

    out = _final_tc(hs[0], hs[1], hs[2],
                    node_graph_ids.reshape(1, N).astype(jnp.int32),
                    head_ids.reshape(B, 1).astype(jnp.int32),
                    tail_ids.reshape(B, 1).astype(jnp.int32),
                    rel_labels.reshape(B, 1).astype(jnp.int32),
                    emb_rel, W_fc, b_fc.reshape(1, 1))
    return out
